# baseline (speedup 1.0000x reference)
import functools

import jax
import jax.numpy as jnp
from jax import lax
from jax.experimental import pallas as pl
from jax.experimental.pallas import tpu as pltpu
from jax.experimental.pallas import tpu_sc as plsc

_NC = 2
_NS = 16
_LANES = 16


def _make_pool(B, L, E):
    NW = _NC * _NS
    BW = B // NW
    HL = L // 2
    CH = 64
    NSTEP = BW // CH
    EG = E // _LANES
    NBUF = 4
    PAIRS = CH // 2
    UNROLL = 5

    mesh = plsc.VectorSubcoreMesh(
        core_axis_name="c", subcore_axis_name="s",
        num_cores=_NC, num_subcores=_NS)

    @functools.partial(
        pl.kernel,
        mesh=mesh,
        compiler_params=pltpu.CompilerParams(use_tc_tiling_on_sc=False),
        out_type=jax.ShapeDtypeStruct((B, E), jnp.float32),
        scratch_types=[
            pltpu.VMEM((2 * CH, HL), jnp.int32),
            pltpu.VMEM((HL, E), jnp.float32),
            pltpu.VMEM((HL, E), jnp.float32),
            pltpu.VMEM((HL, E), jnp.float32),
            pltpu.VMEM((HL, E), jnp.float32),
            pltpu.VMEM((CH, E), jnp.float32),
            pltpu.SemaphoreType.DMA,
            pltpu.SemaphoreType.DMA,
            pltpu.SemaphoreType.DMA,
            pltpu.SemaphoreType.DMA,
        ],
    )
    def pool(ids_hbm, tab_hbm, y_hbm, idx_v, r0, r1, r2, r3, out_v,
             s0, s1, s2, s3):
        bufs = (r0, r1, r2, r3)
        sems = (s0, s1, s2, s3)
        wid = lax.axis_index("s") * _NC + lax.axis_index("c")

        def accumulate(rows, accs):
            def row_body(j, accs):
                a = list(accs)
                for u in range(UNROLL):
                    r = j * UNROLL + u
                    for c in range(EG):
                        a[c] = a[c] + rows[r, pl.ds(_LANES * c, _LANES)]
                return tuple(a)
            return lax.fori_loop(0, HL // UNROLL, row_body, accs)

        for step in range(NSTEP):
            b0 = wid * BW + step * CH
            pltpu.sync_copy(ids_hbm.at[pl.ds(b0 * 2, 2 * CH)], idx_v)
            for q in range(NBUF):
                pltpu.async_copy(tab_hbm.at[idx_v.at[q]], bufs[q], sems[q])

            def pair_body(p, carry):
                for pair in range(2):
                    accs = tuple(jnp.zeros((_LANES,), jnp.float32)
                                 for _ in range(EG))
                    for half in range(2):
                        q = 2 * pair + half
                        h = 4 * p + q
                        pltpu.make_async_copy(
                            tab_hbm.at[idx_v.at[h]], bufs[q], sems[q]
                        ).wait()
                        accs = accumulate(bufs[q], accs)

                        @pl.when(p < PAIRS - 1)
                        def _():
                            pltpu.async_copy(
                                tab_hbm.at[idx_v.at[h + 4]], bufs[q], sems[q])
                    for c in range(EG):
                        out_v[2 * p + pair, pl.ds(_LANES * c, _LANES)] = accs[c]
                return carry

            lax.fori_loop(0, PAIRS, pair_body, 0)
            pltpu.sync_copy(out_v, y_hbm.at[pl.ds(b0, CH)])

    return pool


_TRB = 1024


def _tr_body(x_ref, eye_ref, o_ref):
    t = x_ref[...]
    eye = eye_ref[...]
    dn = (((0,), (0,)), ((), ()))
    o_ref[:, 0:64] = lax.dot_general(
        t[:, 0:_TRB], eye, dn,
        preferred_element_type=jnp.float32,
        precision=lax.Precision.HIGHEST)
    o_ref[:, 64:128] = lax.dot_general(
        t[:, _TRB:2 * _TRB], eye, dn,
        preferred_element_type=jnp.float32,
        precision=lax.Precision.HIGHEST)


def _transpose_pack(embT):
    E, V = embT.shape
    CB = 2 * _TRB
    grid = (V + CB - 1) // CB
    return pl.pallas_call(
        _tr_body,
        grid=(grid,),
        in_specs=[
            pl.BlockSpec((E, CB), lambda i: (0, i)),
            pl.BlockSpec((E, E), lambda i: (0, 0)),
        ],
        out_specs=pl.BlockSpec((_TRB, 2 * E), lambda i: (i, 0)),
        out_shape=jax.ShapeDtypeStruct((grid * _TRB, 2 * E), jnp.float32),
    )(embT, jnp.eye(E, dtype=jnp.float32))


def _mm_body(y_ref, wt_ref, b_ref, o_ref):
    o_ref[...] = jnp.dot(
        y_ref[...], wt_ref[...],
        preferred_element_type=jnp.float32,
        precision=lax.Precision.HIGHEST,
    ) + b_ref[...]


def _matmul(y, Wt, b2):
    B, E = y.shape
    N = Wt.shape[1]
    BB = 1024
    return pl.pallas_call(
        _mm_body,
        grid=(B // BB,),
        in_specs=[
            pl.BlockSpec((BB, E), lambda i: (i, 0)),
            pl.BlockSpec((E, N), lambda i: (0, 0)),
            pl.BlockSpec((1, N), lambda i: (0, 0)),
        ],
        out_specs=pl.BlockSpec((BB, N), lambda i: (i, 0)),
        out_shape=jax.ShapeDtypeStruct((B, N), jnp.float32),
    )(y, Wt, b2)


def kernel(input, embed, W, b):
    B, L = input.shape
    V, E = embed.shape
    v = input.astype(jnp.int32)
    r = (v & ~(2 * _TRB - 1)) + ((v & (_TRB - 1)) << 1) + ((v >> 10) & 1)
    ids2 = r.reshape(2 * B, L // 2)
    packed = _transpose_pack(embed.T)
    table2 = packed.reshape(2 * packed.shape[0], E)
    y = _make_pool(B, L, E)(ids2, table2)
    return _matmul(y, W.T, b.reshape(1, -1))


# speedup vs baseline: 1.4503x; 1.4503x over previous
import functools

import jax
import jax.numpy as jnp
from jax import lax
from jax.experimental import pallas as pl
from jax.experimental.pallas import tpu as pltpu
from jax.experimental.pallas import tpu_sc as plsc

_NC = 2
_NS = 16
_LANES = 16


def _make_pool(B, L, E):
    NW = _NC * _NS
    BW = B // NW
    HL = L // 2
    CH = 64
    NSTEP = BW // CH
    EG = E // _LANES
    NBUF = 8
    GRP = NBUF // 2
    PAIRS = CH // GRP
    UNROLL = 5

    mesh = plsc.VectorSubcoreMesh(
        core_axis_name="c", subcore_axis_name="s",
        num_cores=_NC, num_subcores=_NS)

    @functools.partial(
        pl.kernel,
        mesh=mesh,
        compiler_params=pltpu.CompilerParams(use_tc_tiling_on_sc=False),
        out_type=jax.ShapeDtypeStruct((B, E), jnp.float32),
        scratch_types=[
            pltpu.VMEM((2 * CH, HL), jnp.int32),
        ] + [pltpu.VMEM((HL, E), jnp.float32)] * NBUF
          + [pltpu.VMEM((CH, E), jnp.float32)]
          + [pltpu.SemaphoreType.DMA] * NBUF,
    )
    def pool(ids_hbm, tab_hbm, y_hbm, idx_v, *rest):
        bufs = rest[:NBUF]
        out_v = rest[NBUF]
        sems = rest[NBUF + 1:]
        wid = lax.axis_index("s") * _NC + lax.axis_index("c")

        def accumulate(rows, accs):
            def row_body(j, accs):
                a = list(accs)
                for u in range(UNROLL):
                    r = j * UNROLL + u
                    for c in range(EG):
                        a[c] = a[c] + rows[r, pl.ds(_LANES * c, _LANES)]
                return tuple(a)
            return lax.fori_loop(0, HL // UNROLL, row_body, accs)

        for step in range(NSTEP):
            b0 = wid * BW + step * CH
            pltpu.sync_copy(ids_hbm.at[pl.ds(b0 * 2, 2 * CH)], idx_v)
            for q in range(NBUF):
                pltpu.async_copy(tab_hbm.at[idx_v.at[q]], bufs[q], sems[q])

            def pair_body(p, carry):
                for pair in range(GRP):
                    accs = tuple(jnp.zeros((_LANES,), jnp.float32)
                                 for _ in range(EG))
                    for half in range(2):
                        q = 2 * pair + half
                        h = NBUF * p + q
                        pltpu.make_async_copy(
                            tab_hbm.at[idx_v.at[h]], bufs[q], sems[q]
                        ).wait()
                        accs = accumulate(bufs[q], accs)

                        @pl.when(p < PAIRS - 1)
                        def _():
                            pltpu.async_copy(
                                tab_hbm.at[idx_v.at[h + NBUF]],
                                bufs[q], sems[q])
                    for c in range(EG):
                        out_v[GRP * p + pair,
                              pl.ds(_LANES * c, _LANES)] = accs[c]
                return carry

            lax.fori_loop(0, PAIRS, pair_body, 0)
            pltpu.sync_copy(out_v, y_hbm.at[pl.ds(b0, CH)])

    return pool


def _tr_body(x_ref, o_ref):
    o_ref[:, 0:64] = x_ref[...].T


def _transpose_pack(embT):
    E, V = embT.shape
    TB = 4096
    grid = (V + TB - 1) // TB
    return pl.pallas_call(
        _tr_body,
        grid=(grid,),
        in_specs=[pl.BlockSpec((E, TB), lambda i: (0, i))],
        out_specs=pl.BlockSpec((TB, 2 * E), lambda i: (i, 0)),
        out_shape=jax.ShapeDtypeStruct((V, 2 * E), jnp.float32),
    )(embT)


def _mm_body(y_ref, wt_ref, b_ref, o_ref):
    o_ref[...] = jnp.dot(
        y_ref[...], wt_ref[...],
        preferred_element_type=jnp.float32,
        precision=lax.Precision.HIGHEST,
    ) + b_ref[...]


def _matmul(y, Wt, b2):
    B, E = y.shape
    N = Wt.shape[1]
    BB = 1024
    return pl.pallas_call(
        _mm_body,
        grid=(B // BB,),
        in_specs=[
            pl.BlockSpec((BB, E), lambda i: (i, 0)),
            pl.BlockSpec((E, N), lambda i: (0, 0)),
            pl.BlockSpec((1, N), lambda i: (0, 0)),
        ],
        out_specs=pl.BlockSpec((BB, N), lambda i: (i, 0)),
        out_shape=jax.ShapeDtypeStruct((B, N), jnp.float32),
    )(y, Wt, b2)


def kernel(input, embed, W, b):
    B, L = input.shape
    V, E = embed.shape
    ids2 = (input.astype(jnp.int32) * 2).reshape(2 * B, L // 2)
    packed = _transpose_pack(embed.T)
    table2 = packed.reshape(2 * packed.shape[0], E)
    y = _make_pool(B, L, E)(ids2, table2)
    return _matmul(y, W.T, b.reshape(1, -1))
